# baseline (speedup 1.0000x reference)
import jax
import jax.numpy as jnp
from jax.experimental import pallas as pl
from jax.experimental.pallas import tpu as pltpu

H = 128


def _gru_body(x_ref, wih_ref, whh_ref, bih_ref, bhh_ref, o_ref,
              w_ref, b_ref, ones_ref):
    i = pl.program_id(0)

    @pl.when(i == 0)
    def _init():
        wih = wih_ref[...]
        whh = whh_ref[...]
        w_rz = 0.5 * (wih[0:2 * H] + whh[0:2 * H])
        w_ref[:, 0:2 * H] = w_rz.T
        w_ref[:, 2 * H:3 * H] = wih[2 * H:3 * H].T
        w_ref[:, 3 * H:4 * H] = whh[2 * H:3 * H].T
        bih = bih_ref[...][None, :]
        bhh = bhh_ref[...][None, :]
        b_ref[:, 0:2 * H] = 0.5 * (bih[:, 0:2 * H] + bhh[:, 0:2 * H])
        b_ref[:, 2 * H:3 * H] = bih[:, 2 * H:3 * H]
        b_ref[:, 3 * H:4 * H] = bhh[:, 2 * H:3 * H]
        ones_ref[...] = jnp.ones((H, H), jnp.float32)

    x = x_ref[...]
    s = jnp.dot(x * x, ones_ref[...], preferred_element_type=jnp.float32)
    h = x * jax.lax.rsqrt(jnp.maximum(s, 1e-24))
    g = jnp.dot(h, w_ref[...], preferred_element_type=jnp.float32) + b_ref[...]
    r = 0.5 * (1.0 + jnp.tanh(g[:, 0:H]))
    z = 0.5 * (1.0 + jnp.tanh(g[:, H:2 * H]))
    c = jnp.tanh(g[:, 2 * H:3 * H] + r * g[:, 3 * H:4 * H])
    o_ref[...] = c + z * (h - c)


def kernel(dynamic_emb, emb_rel, W_ih_rel, W_hh_rel, b_ih_rel, b_hh_rel,
           W_ih_ent, W_hh_ent, b_ih_ent, b_hh_ent, r_to_e, seg_ids):
    N, Hd = dynamic_emb.shape
    B = 1000
    out = pl.pallas_call(
        _gru_body,
        grid=(N // B,),
        in_specs=[
            pl.BlockSpec((B, Hd), lambda i: (i, 0)),
            pl.BlockSpec((3 * H, Hd), lambda i: (0, 0)),
            pl.BlockSpec((3 * H, Hd), lambda i: (0, 0)),
            pl.BlockSpec((3 * H,), lambda i: (0,)),
            pl.BlockSpec((3 * H,), lambda i: (0,)),
        ],
        out_specs=pl.BlockSpec((B, Hd), lambda i: (i, 0)),
        out_shape=jax.ShapeDtypeStruct((N, Hd), jnp.float32),
        scratch_shapes=[
            pltpu.VMEM((Hd, 4 * H), jnp.float32),
            pltpu.VMEM((1, 4 * H), jnp.float32),
            pltpu.VMEM((H, H), jnp.float32),
        ],
        compiler_params=pltpu.CompilerParams(
            dimension_semantics=("parallel",)),
    )(dynamic_emb, W_ih_ent, W_hh_ent, b_ih_ent, b_hh_ent)
    return out


# speedup vs baseline: 1.3260x; 1.3260x over previous
import jax
import jax.numpy as jnp
from jax.experimental import pallas as pl
from jax.experimental.pallas import tpu as pltpu

H = 128


def _gru_body(x_ref, wih_ref, whh_ref, bih_ref, bhh_ref, o_ref,
              w_ref, b_ref, ones_ref):
    i = pl.program_id(0)

    @pl.when(i == 0)
    def _init():
        wih = wih_ref[...]
        whh = whh_ref[...]
        w_rz = 0.5 * (wih[0:2 * H] + whh[0:2 * H])
        w_ref[:, 0:2 * H] = w_rz.T
        w_ref[:, 2 * H:3 * H] = wih[2 * H:3 * H].T
        w_ref[:, 3 * H:4 * H] = whh[2 * H:3 * H].T
        bih = bih_ref[...][None, :]
        bhh = bhh_ref[...][None, :]
        b_ref[:, 0:2 * H] = 0.5 * (bih[:, 0:2 * H] + bhh[:, 0:2 * H])
        b_ref[:, 2 * H:3 * H] = bih[:, 2 * H:3 * H]
        b_ref[:, 3 * H:4 * H] = bhh[:, 2 * H:3 * H]
        ones_ref[...] = jnp.ones((H, H), jnp.float32)

    x = x_ref[...]
    s = jnp.dot(x * x, ones_ref[...], preferred_element_type=jnp.float32)
    h = x * jax.lax.rsqrt(jnp.maximum(s, 1e-24))
    g = jnp.dot(h, w_ref[...], preferred_element_type=jnp.float32) + b_ref[...]
    r = 0.5 * (1.0 + jnp.tanh(g[:, 0:H]))
    z = 0.5 * (1.0 + jnp.tanh(g[:, H:2 * H]))
    c = jnp.tanh(g[:, 2 * H:3 * H] + r * g[:, 3 * H:4 * H])
    o_ref[...] = c + z * (h - c)


def kernel(dynamic_emb, emb_rel, W_ih_rel, W_hh_rel, b_ih_rel, b_hh_rel,
           W_ih_ent, W_hh_ent, b_ih_ent, b_hh_ent, r_to_e, seg_ids):
    N, Hd = dynamic_emb.shape
    B = 5000
    out = pl.pallas_call(
        _gru_body,
        grid=(N // B,),
        in_specs=[
            pl.BlockSpec((B, Hd), lambda i: (i, 0)),
            pl.BlockSpec((3 * H, Hd), lambda i: (0, 0)),
            pl.BlockSpec((3 * H, Hd), lambda i: (0, 0)),
            pl.BlockSpec((3 * H,), lambda i: (0,)),
            pl.BlockSpec((3 * H,), lambda i: (0,)),
        ],
        out_specs=pl.BlockSpec((B, Hd), lambda i: (i, 0)),
        out_shape=jax.ShapeDtypeStruct((N, Hd), jnp.float32),
        scratch_shapes=[
            pltpu.VMEM((Hd, 4 * H), jnp.float32),
            pltpu.VMEM((1, 4 * H), jnp.float32),
            pltpu.VMEM((H, H), jnp.float32),
        ],
        compiler_params=pltpu.CompilerParams(
            dimension_semantics=("parallel",)),
    )(dynamic_emb, W_ih_ent, W_hh_ent, b_ih_ent, b_hh_ent)
    return out
